# Initial kernel scaffold; baseline (speedup 1.0000x reference)
import jax
import jax.numpy as jnp
from jax.experimental import pallas as pl


def kernel(x, rbf, neighbors, W1, b1, W2, b2):
    raise NotImplementedError("write your pallas kernel here")


# speedup vs baseline: 4.0917x; 4.0917x over previous
import functools

import jax
import jax.numpy as jnp
from jax import lax
from jax.experimental import pallas as pl
from jax.experimental.pallas import tpu as pltpu
from jax.experimental.pallas import tpu_sc as plsc

N = 10000
K = 64
F = 128
R = 16
E = N * K

NC = 2
NS = 16
NW = NC * NS
EPW = E // NW
CH = 80
CPW = EPW // CH


def _gather_body(x_hbm, nb_hbm, out_hbm, idx_v, rows0, rows1, sem0, sem1):
    wid = lax.axis_index("s") * NC + lax.axis_index("c")
    base = wid * EPW
    pltpu.sync_copy(nb_hbm.at[wid], idx_v)
    pltpu.async_copy(x_hbm.at[idx_v.at[0]], rows0, sem0)

    def body(jj, carry):
        j = jj * 2
        pltpu.make_async_copy(x_hbm.at[idx_v.at[j]], rows0, sem0).wait()
        pltpu.async_copy(x_hbm.at[idx_v.at[j + 1]], rows1, sem1)
        pltpu.sync_copy(rows0, out_hbm.at[pl.ds(base + j * CH, CH)])
        pltpu.make_async_copy(x_hbm.at[idx_v.at[j + 1]], rows1, sem1).wait()

        @pl.when(jj < CPW // 2 - 1)
        def _():
            pltpu.async_copy(x_hbm.at[idx_v.at[j + 2]], rows0, sem0)

        pltpu.sync_copy(rows1, out_hbm.at[pl.ds(base + (j + 1) * CH, CH)])
        return carry

    lax.fori_loop(0, CPW // 2, body, 0)


@functools.cache
def _sc_gather_kernel():
    return pl.kernel(
        _gather_body,
        out_type=jax.ShapeDtypeStruct((E, F), jnp.float32),
        mesh=plsc.VectorSubcoreMesh(
            core_axis_name="c", subcore_axis_name="s", num_cores=NC, num_subcores=NS
        ),
        scratch_types=[
            pltpu.VMEM((CPW, CH), jnp.int32),
            pltpu.VMEM((CH, F), jnp.float32),
            pltpu.VMEM((CH, F), jnp.float32),
            pltpu.SemaphoreType.DMA,
            pltpu.SemaphoreType.DMA,
        ],
    )


TN = 200
GRID = N // TN


def _tc_body(rbf_ref, xj_ref, w1_ref, b1_ref, w2_ref, b2_ref, out_ref):
    rbf2 = rbf_ref[...].reshape(TN * K, R)
    h = jnp.dot(rbf2, w1_ref[...], preferred_element_type=jnp.float32)
    h = h + b1_ref[...]
    h = jnp.maximum(h, 0.0) + jnp.log1p(jnp.exp(-jnp.abs(h)))
    w = jnp.dot(h, w2_ref[...], preferred_element_type=jnp.float32)
    w = w + b2_ref[...]
    prod = xj_ref[...] * w
    out_ref[...] = prod.reshape(TN, K, F).sum(axis=1)


def _tc_cfconv(rbf, xj, W1, b1, W2, b2):
    return pl.pallas_call(
        _tc_body,
        grid=(GRID,),
        in_specs=[
            pl.BlockSpec((TN, K, R), lambda i: (i, 0, 0)),
            pl.BlockSpec((TN * K, F), lambda i: (i, 0)),
            pl.BlockSpec((R, F), lambda i: (0, 0)),
            pl.BlockSpec((1, F), lambda i: (0, 0)),
            pl.BlockSpec((F, F), lambda i: (0, 0)),
            pl.BlockSpec((1, F), lambda i: (0, 0)),
        ],
        out_specs=pl.BlockSpec((TN, F), lambda i: (i, 0)),
        out_shape=jax.ShapeDtypeStruct((N, F), jnp.float32),
    )(rbf, xj, W1, b1, W2, b2)


def kernel(x, rbf, neighbors, W1, b1, W2, b2):
    nb = neighbors.astype(jnp.int32).reshape(NW, CPW, CH)
    xj = _sc_gather_kernel()(x, nb)
    return _tc_cfconv(rbf, xj, W1, b1.reshape(1, F), W2, b2.reshape(1, F))
